# Initial kernel scaffold; baseline (speedup 1.0000x reference)
import jax
import jax.numpy as jnp
from jax.experimental import pallas as pl


def kernel(boxes, scores):
    raise NotImplementedError("write your pallas kernel here")


# speedup vs baseline: 230.7547x; 230.7547x over previous
import jax
import jax.numpy as jnp
from jax import lax
from jax.experimental import pallas as pl
from jax.experimental.pallas import tpu as pltpu
from jax.experimental.pallas import tpu_sc as plsc

_N = 5000
_MAX_DET = 100
_IOU_THRESH = 0.5
_SCORE_THRESH = 0.05
_L = 16
_KSLOTS = 112
_STAGE1 = 256


def _splat(v):
    return jnp.full((_L,), v, jnp.int32)


def _nms_body(boxes_hbm, scores_hbm, order_hbm, out_hbm,
              boxes_v, scores_v, order_v,
              kx0, ky0, kx1, ky1, ka, supf, out_v):
    cid = lax.axis_index("c")
    sid = lax.axis_index("s")

    pltpu.sync_copy(boxes_hbm, boxes_v)
    pltpu.sync_copy(scores_hbm, scores_v)
    pltpu.sync_copy(order_hbm, order_v)

    neg = jnp.full((_L,), -1e6, jnp.float32)
    zero = jnp.zeros((_L,), jnp.float32)
    for k in range(_KSLOTS // _L):
        sl = pl.ds(k * _L, _L)
        kx0[sl] = neg
        ky0[sl] = neg
        kx1[sl] = neg
        ky1[sl] = neg
        ka[sl] = zero

    lane0 = lax.iota(jnp.int32, _L) == 0

    def load_box(i):
        ov = plsc.load_gather(order_v, [_splat(0) + i])
        ov4 = ov * 4
        x0 = plsc.load_gather(boxes_v, [ov4])
        y0 = plsc.load_gather(boxes_v, [ov4 + 1])
        x1 = plsc.load_gather(boxes_v, [ov4 + 2])
        y1 = plsc.load_gather(boxes_v, [ov4 + 3])
        s = plsc.load_gather(scores_v, [ov])
        return x0, y0, x1, y1, s

    def emit_row(k, x0, y0, x1, y1, s, mask):
        base = _splat(0) + k * 5
        plsc.store_scatter(out_v, [base], x0, mask=mask)
        plsc.store_scatter(out_v, [base + 1], y0, mask=mask)
        plsc.store_scatter(out_v, [base + 2], x1, mask=mask)
        plsc.store_scatter(out_v, [base + 3], y1, mask=mask)
        plsc.store_scatter(out_v, [base + 4], s, mask=mask)

    def scan_step(i, k):
        done = k >= _MAX_DET

        @pl.when(jnp.logical_not(done))
        def _step():
            x0, y0, x1, y1, s = load_box(i)
            area = (x1 - x0) * (y1 - y0)
            maxiou = jnp.zeros((_L,), jnp.float32)
            for kk in range(_KSLOTS // _L):
                sl = pl.ds(kk * _L, _L)
                bx0 = kx0[sl]
                by0 = ky0[sl]
                bx1 = kx1[sl]
                by1 = ky1[sl]
                ba = ka[sl]
                ltx = jnp.maximum(x0, bx0)
                lty = jnp.maximum(y0, by0)
                rbx = jnp.minimum(x1, bx1)
                rby = jnp.minimum(y1, by1)
                w = jnp.maximum(rbx - ltx, jnp.float32(0.0))
                h = jnp.maximum(rby - lty, jnp.float32(0.0))
                inter = w * h
                iou = inter / ((area + ba) - inter + jnp.float32(1e-9))
                maxiou = jnp.maximum(maxiou, iou)
            m = jnp.max(maxiou)
            smax = jnp.max(s)
            suppressed = (smax < _SCORE_THRESH) | (m > _IOU_THRESH)
            sup_i32 = suppressed.astype(jnp.int32)

            keep_mask = lane0 & jnp.logical_not(suppressed)
            kv = _splat(0) + k
            plsc.store_scatter(kx0, [kv], x0, mask=keep_mask)
            plsc.store_scatter(ky0, [kv], y0, mask=keep_mask)
            plsc.store_scatter(kx1, [kv], x1, mask=keep_mask)
            plsc.store_scatter(ky1, [kv], y1, mask=keep_mask)
            plsc.store_scatter(ka, [kv], area, mask=keep_mask)
            emit_row(k, x0, y0, x1, y1, s, keep_mask)

            plsc.store_scatter(supf, [_splat(0) + i],
                               _splat(0) + sup_i32, mask=lane0)

        sup_rb = jnp.max(plsc.load_gather(supf, [_splat(0) + i]))
        return k + jnp.where(done, 0, 1 - sup_rb)

    k_end = lax.fori_loop(0, _STAGE1, scan_step, jnp.int32(0))

    @pl.when(k_end < _MAX_DET)
    def _slow_tail():
        k_mid = lax.fori_loop(_STAGE1, _N, scan_step, k_end)

        def fill_step(j, k):
            fdone = k >= _MAX_DET
            sj = jnp.max(plsc.load_gather(supf, [_splat(0) + j]))
            x0, y0, x1, y1, s = load_box(j)
            emit = lane0 & (sj == 1) & jnp.logical_not(fdone)
            emit_row(k, x0, y0, x1, y1, s, emit)
            return k + jnp.where(fdone, 0, sj)

        lax.fori_loop(0, _N, fill_step, k_mid)

    @pl.when((cid == 0) & (sid == 0))
    def _store():
        pltpu.sync_copy(out_v, out_hbm)


def _make_nms():
    mesh = plsc.VectorSubcoreMesh(core_axis_name="c", subcore_axis_name="s",
                                  num_cores=2, num_subcores=16)
    return pl.kernel(
        _nms_body,
        out_type=jax.ShapeDtypeStruct((_MAX_DET * 5,), jnp.float32),
        mesh=mesh,
        scratch_types=[
            pltpu.VMEM((_N * 4,), jnp.float32),
            pltpu.VMEM((_N,), jnp.float32),
            pltpu.VMEM((_N,), jnp.int32),
            pltpu.VMEM((_KSLOTS,), jnp.float32),
            pltpu.VMEM((_KSLOTS,), jnp.float32),
            pltpu.VMEM((_KSLOTS,), jnp.float32),
            pltpu.VMEM((_KSLOTS,), jnp.float32),
            pltpu.VMEM((_KSLOTS,), jnp.float32),
            pltpu.VMEM((_N,), jnp.int32),
            pltpu.VMEM((_MAX_DET * 5,), jnp.float32),
        ],
        compiler_params=pltpu.CompilerParams(needs_layout_passes=False),
        name="sc_greedy_nms",
    )


def kernel(boxes, scores):
    order = jnp.argsort(-scores).astype(jnp.int32)
    flat = _make_nms()(boxes.reshape(-1), scores, order)
    return flat.reshape(_MAX_DET, 5)


# speedup vs baseline: 250.7660x; 1.0867x over previous
import jax
import jax.numpy as jnp
from jax import lax
from jax.experimental import pallas as pl
from jax.experimental.pallas import tpu as pltpu
from jax.experimental.pallas import tpu_sc as plsc

_N = 5000
_MAX_DET = 100
_IOU_THRESH = 0.5
_SCORE_THRESH = 0.05
_L = 16
_KSLOTS = 112
_STAGE1 = 256


def _splat(v):
    return jnp.full((_L,), v, jnp.int32)


def _nms_body(boxes_hbm, scores_hbm, order_hbm, out_hbm,
              boxes_v, scores_v, order_v,
              kx0, ky0, kx1, ky1, ka, supf, out_v):
    cid = lax.axis_index("c")
    sid = lax.axis_index("s")

    pltpu.sync_copy(boxes_hbm, boxes_v)
    pltpu.sync_copy(scores_hbm, scores_v)
    pltpu.sync_copy(order_hbm, order_v)

    neg = jnp.full((_L,), -1e6, jnp.float32)
    zero = jnp.zeros((_L,), jnp.float32)
    for k in range(_KSLOTS // _L):
        sl = pl.ds(k * _L, _L)
        kx0[sl] = neg
        ky0[sl] = neg
        kx1[sl] = neg
        ky1[sl] = neg
        ka[sl] = zero

    lane0 = lax.iota(jnp.int32, _L) == 0

    def load_box(i):
        ov = plsc.load_gather(order_v, [_splat(0) + i])
        ov4 = ov * 4
        x0 = plsc.load_gather(boxes_v, [ov4])
        y0 = plsc.load_gather(boxes_v, [ov4 + 1])
        x1 = plsc.load_gather(boxes_v, [ov4 + 2])
        y1 = plsc.load_gather(boxes_v, [ov4 + 3])
        s = plsc.load_gather(scores_v, [ov])
        return x0, y0, x1, y1, s

    def emit_row(k, x0, y0, x1, y1, s, mask):
        base = _splat(0) + k * 5
        plsc.store_scatter(out_v, [base], x0, mask=mask)
        plsc.store_scatter(out_v, [base + 1], y0, mask=mask)
        plsc.store_scatter(out_v, [base + 2], x1, mask=mask)
        plsc.store_scatter(out_v, [base + 3], y1, mask=mask)
        plsc.store_scatter(out_v, [base + 4], s, mask=mask)

    def scan_step(i, k):
        done = k >= _MAX_DET

        @pl.when(jnp.logical_not(done))
        def _step():
            x0, y0, x1, y1, s = load_box(i)
            area = (x1 - x0) * (y1 - y0)
            maxiou = jnp.zeros((_L,), jnp.float32)
            for kk in range(_KSLOTS // _L):
                sl = pl.ds(kk * _L, _L)
                bx0 = kx0[sl]
                by0 = ky0[sl]
                bx1 = kx1[sl]
                by1 = ky1[sl]
                ba = ka[sl]
                ltx = jnp.maximum(x0, bx0)
                lty = jnp.maximum(y0, by0)
                rbx = jnp.minimum(x1, bx1)
                rby = jnp.minimum(y1, by1)
                w = jnp.maximum(rbx - ltx, jnp.float32(0.0))
                h = jnp.maximum(rby - lty, jnp.float32(0.0))
                inter = w * h
                iou = inter / ((area + ba) - inter + jnp.float32(1e-9))
                maxiou = jnp.maximum(maxiou, iou)
            m = jnp.max(maxiou)
            smax = jnp.max(s)
            suppressed = (smax < _SCORE_THRESH) | (m > _IOU_THRESH)
            sup_i32 = suppressed.astype(jnp.int32)

            keep_mask = lane0 & jnp.logical_not(suppressed)
            kv = _splat(0) + k
            plsc.store_scatter(kx0, [kv], x0, mask=keep_mask)
            plsc.store_scatter(ky0, [kv], y0, mask=keep_mask)
            plsc.store_scatter(kx1, [kv], x1, mask=keep_mask)
            plsc.store_scatter(ky1, [kv], y1, mask=keep_mask)
            plsc.store_scatter(ka, [kv], area, mask=keep_mask)
            emit_row(k, x0, y0, x1, y1, s, keep_mask)

            plsc.store_scatter(supf, [_splat(0) + i],
                               _splat(0) + sup_i32, mask=lane0)

        sup_rb = jnp.max(plsc.load_gather(supf, [_splat(0) + i]))
        return k + jnp.where(done, 0, 1 - sup_rb)

    k_end = lax.fori_loop(0, _STAGE1, scan_step, jnp.int32(0))

    @pl.when(k_end < _MAX_DET)
    def _slow_tail():
        k_mid = lax.fori_loop(_STAGE1, _N, scan_step, k_end)

        def fill_step(j, k):
            fdone = k >= _MAX_DET
            sj = jnp.max(plsc.load_gather(supf, [_splat(0) + j]))
            x0, y0, x1, y1, s = load_box(j)
            emit = lane0 & (sj == 1) & jnp.logical_not(fdone)
            emit_row(k, x0, y0, x1, y1, s, emit)
            return k + jnp.where(fdone, 0, sj)

        lax.fori_loop(0, _N, fill_step, k_mid)

    @pl.when((cid == 0) & (sid == 0))
    def _store():
        pltpu.sync_copy(out_v, out_hbm)


def _make_nms():
    mesh = plsc.VectorSubcoreMesh(core_axis_name="c", subcore_axis_name="s",
                                  num_cores=1, num_subcores=16)
    return pl.kernel(
        _nms_body,
        out_type=jax.ShapeDtypeStruct((_MAX_DET * 5,), jnp.float32),
        mesh=mesh,
        scratch_types=[
            pltpu.VMEM((_N * 4,), jnp.float32),
            pltpu.VMEM((_N,), jnp.float32),
            pltpu.VMEM((_N,), jnp.int32),
            pltpu.VMEM((_KSLOTS,), jnp.float32),
            pltpu.VMEM((_KSLOTS,), jnp.float32),
            pltpu.VMEM((_KSLOTS,), jnp.float32),
            pltpu.VMEM((_KSLOTS,), jnp.float32),
            pltpu.VMEM((_KSLOTS,), jnp.float32),
            pltpu.VMEM((_N,), jnp.int32),
            pltpu.VMEM((_MAX_DET * 5,), jnp.float32),
        ],
        compiler_params=pltpu.CompilerParams(needs_layout_passes=False),
        name="sc_greedy_nms",
    )


def kernel(boxes, scores):
    order = jnp.argsort(-scores).astype(jnp.int32)
    flat = _make_nms()(boxes.reshape(-1), scores, order)
    return flat.reshape(_MAX_DET, 5)


# speedup vs baseline: 298.6142x; 1.1908x over previous
import jax
import jax.numpy as jnp
from jax import lax
from jax.experimental import pallas as pl
from jax.experimental.pallas import tpu as pltpu
from jax.experimental.pallas import tpu_sc as plsc

_N = 5000
_MAX_DET = 100
_IOU_THRESH = 0.5
_SCORE_THRESH = 0.05
_L = 16
_KSLOTS = 112
_STAGE1 = 128


def _splat(v):
    return jnp.full((_L,), v, jnp.int32)


def _nms_body(boxes_hbm, scores_hbm, order_hbm, out_hbm,
              boxes_v, scores_v, order_v,
              kx0, ky0, kx1, ky1, ka, supf, out_v):
    cid = lax.axis_index("c")
    sid = lax.axis_index("s")

    pltpu.sync_copy(boxes_hbm, boxes_v)
    pltpu.sync_copy(scores_hbm, scores_v)
    pltpu.sync_copy(order_hbm, order_v)

    neg = jnp.full((_L,), -1e6, jnp.float32)
    zero = jnp.zeros((_L,), jnp.float32)
    for k in range(_KSLOTS // _L):
        sl = pl.ds(k * _L, _L)
        kx0[sl] = neg
        ky0[sl] = neg
        kx1[sl] = neg
        ky1[sl] = neg
        ka[sl] = zero

    lane0 = lax.iota(jnp.int32, _L) == 0

    def load_box(i):
        ov = plsc.load_gather(order_v, [_splat(0) + i])
        ov4 = ov * 4
        x0 = plsc.load_gather(boxes_v, [ov4])
        y0 = plsc.load_gather(boxes_v, [ov4 + 1])
        x1 = plsc.load_gather(boxes_v, [ov4 + 2])
        y1 = plsc.load_gather(boxes_v, [ov4 + 3])
        s = plsc.load_gather(scores_v, [ov])
        return x0, y0, x1, y1, s

    def emit_row(kv, x0, y0, x1, y1, s, mask):
        base = kv * 5
        plsc.store_scatter(out_v, [base], x0, mask=mask)
        plsc.store_scatter(out_v, [base + 1], y0, mask=mask)
        plsc.store_scatter(out_v, [base + 2], x1, mask=mask)
        plsc.store_scatter(out_v, [base + 3], y1, mask=mask)
        plsc.store_scatter(out_v, [base + 4], s, mask=mask)

    def max_iou_vs_kept(x0, y0, x1, y1, area):
        maxiou = jnp.zeros((_L,), jnp.float32)
        for kk in range(_KSLOTS // _L):
            sl = pl.ds(kk * _L, _L)
            bx0 = kx0[sl]
            by0 = ky0[sl]
            bx1 = kx1[sl]
            by1 = ky1[sl]
            ba = ka[sl]
            ltx = jnp.maximum(x0, bx0)
            lty = jnp.maximum(y0, by0)
            rbx = jnp.minimum(x1, bx1)
            rby = jnp.minimum(y1, by1)
            w = jnp.maximum(rbx - ltx, jnp.float32(0.0))
            h = jnp.maximum(rby - lty, jnp.float32(0.0))
            inter = w * h
            iou = inter / ((area + ba) - inter + jnp.float32(1e-9))
            maxiou = jnp.maximum(maxiou, iou)
        return maxiou

    def keep_candidate(kv, x0, y0, x1, y1, s, area, keep_mask):
        plsc.store_scatter(kx0, [kv], x0, mask=keep_mask)
        plsc.store_scatter(ky0, [kv], y0, mask=keep_mask)
        plsc.store_scatter(kx1, [kv], x1, mask=keep_mask)
        plsc.store_scatter(ky1, [kv], y1, mask=keep_mask)
        plsc.store_scatter(ka, [kv], area, mask=keep_mask)
        emit_row(kv, x0, y0, x1, y1, s, keep_mask)

    def scan_step_fast(i, kvec):
        x0, y0, x1, y1, s = load_box(i)
        area = (x1 - x0) * (y1 - y0)
        maxiou = max_iou_vs_kept(x0, y0, x1, y1, area)
        cnt = plsc.all_reduce_population_count(maxiou > _IOU_THRESH)
        sup_vec = (cnt > 0) | (s < _SCORE_THRESH)
        sup_i32 = sup_vec.astype(jnp.int32)
        notdone = kvec < _MAX_DET

        keep_candidate(kvec, x0, y0, x1, y1, s, area,
                       lane0 & jnp.logical_not(sup_vec) & notdone)
        plsc.store_scatter(supf, [_splat(0) + i], sup_i32,
                           mask=lane0 & notdone)
        return kvec + jnp.where(notdone, 1 - sup_i32, 0)

    def scan_step(i, k):
        done = k >= _MAX_DET

        @pl.when(jnp.logical_not(done))
        def _step():
            x0, y0, x1, y1, s = load_box(i)
            area = (x1 - x0) * (y1 - y0)
            maxiou = max_iou_vs_kept(x0, y0, x1, y1, area)
            cnt = plsc.all_reduce_population_count(maxiou > _IOU_THRESH)
            sup_vec = (cnt > 0) | (s < _SCORE_THRESH)
            sup_i32 = sup_vec.astype(jnp.int32)
            keep_candidate(_splat(0) + k, x0, y0, x1, y1, s, area,
                           lane0 & jnp.logical_not(sup_vec))
            plsc.store_scatter(supf, [_splat(0) + i], sup_i32, mask=lane0)

        sup_rb = jnp.max(plsc.load_gather(supf, [_splat(0) + i]))
        return k + jnp.where(done, 0, 1 - sup_rb)

    kvec_end = lax.fori_loop(0, _STAGE1, scan_step_fast,
                             jnp.zeros((_L,), jnp.int32))
    k_end = jnp.max(kvec_end)

    @pl.when(k_end < _MAX_DET)
    def _slow_tail():
        k_mid = lax.fori_loop(_STAGE1, _N, scan_step, k_end)

        def fill_step(j, k):
            fdone = k >= _MAX_DET
            sj = jnp.max(plsc.load_gather(supf, [_splat(0) + j]))
            x0, y0, x1, y1, s = load_box(j)
            emit = lane0 & (sj == 1) & jnp.logical_not(fdone)
            emit_row(_splat(0) + k, x0, y0, x1, y1, s, emit)
            return k + jnp.where(fdone, 0, sj)

        lax.fori_loop(0, _N, fill_step, k_mid)

    @pl.when((cid == 0) & (sid == 0))
    def _store():
        pltpu.sync_copy(out_v, out_hbm)


def _make_nms():
    mesh = plsc.VectorSubcoreMesh(core_axis_name="c", subcore_axis_name="s",
                                  num_cores=1, num_subcores=16)
    return pl.kernel(
        _nms_body,
        out_type=jax.ShapeDtypeStruct((_MAX_DET * 5,), jnp.float32),
        mesh=mesh,
        scratch_types=[
            pltpu.VMEM((_N * 4,), jnp.float32),
            pltpu.VMEM((_N,), jnp.float32),
            pltpu.VMEM((_N,), jnp.int32),
            pltpu.VMEM((_KSLOTS,), jnp.float32),
            pltpu.VMEM((_KSLOTS,), jnp.float32),
            pltpu.VMEM((_KSLOTS,), jnp.float32),
            pltpu.VMEM((_KSLOTS,), jnp.float32),
            pltpu.VMEM((_KSLOTS,), jnp.float32),
            pltpu.VMEM((_N,), jnp.int32),
            pltpu.VMEM((_MAX_DET * 5,), jnp.float32),
        ],
        compiler_params=pltpu.CompilerParams(needs_layout_passes=False),
        name="sc_greedy_nms",
    )


def kernel(boxes, scores):
    order = jnp.argsort(-scores).astype(jnp.int32)
    flat = _make_nms()(boxes.reshape(-1), scores, order)
    return flat.reshape(_MAX_DET, 5)
